# Initial kernel scaffold; baseline (speedup 1.0000x reference)
import jax
import jax.numpy as jnp
from jax.experimental import pallas as pl


def kernel(edge_index, edge_type, edge_pairs, node_emb, basis1, comp1, root1, bias1, basis2, comp2, root2, bias2, lin1_w, lin1_b, lin2_w, lin2_b):
    raise NotImplementedError("write your pallas kernel here")


# speedup vs baseline: 5.1992x; 5.1992x over previous
import functools

import jax
import jax.numpy as jnp
from jax.experimental import pallas as pl

_N_NODES = 50000
_N_REL = 8


def _layer_body(a_ref, inv_ref, x_ref, w_ref, root_ref, bias_ref, o_ref, *, relu):
    a = a_ref[...] * inv_ref[...]
    out = jnp.dot(a, w_ref[...], preferred_element_type=jnp.float32)
    out = out + jnp.dot(x_ref[...], root_ref[...], preferred_element_type=jnp.float32)
    out = out + bias_ref[...]
    if relu:
        out = jnp.maximum(out, 0.0)
    o_ref[...] = out


def _rgcn_layer(x, src, dst, etype, basis, comp, root, bias, relu):
    emb = x.shape[1]
    hid = basis.shape[2]
    w = jnp.einsum('rb,bio->rio', comp, basis).reshape(_N_REL * emb, hid)
    seg = dst * _N_REL + etype
    msgs = jnp.take(x, src, axis=0)
    acc = jax.ops.segment_sum(msgs, seg, num_segments=_N_NODES * _N_REL)
    cnt = jax.ops.segment_sum(jnp.ones((src.shape[0],), jnp.float32), seg,
                              num_segments=_N_NODES * _N_REL)
    acc = acc.reshape(_N_NODES, _N_REL * emb)
    inv = (1.0 / jnp.maximum(cnt, 1.0)).reshape(_N_NODES, _N_REL)
    inv = jnp.repeat(inv, emb, axis=1)

    bn = 1000
    grid = _N_NODES // bn
    return pl.pallas_call(
        functools.partial(_layer_body, relu=relu),
        grid=(grid,),
        in_specs=[
            pl.BlockSpec((bn, _N_REL * emb), lambda i: (i, 0)),
            pl.BlockSpec((bn, _N_REL * emb), lambda i: (i, 0)),
            pl.BlockSpec((bn, emb), lambda i: (i, 0)),
            pl.BlockSpec((_N_REL * emb, hid), lambda i: (0, 0)),
            pl.BlockSpec((emb, hid), lambda i: (0, 0)),
            pl.BlockSpec((1, hid), lambda i: (0, 0)),
        ],
        out_specs=pl.BlockSpec((bn, hid), lambda i: (i, 0)),
        out_shape=jax.ShapeDtypeStruct((_N_NODES, hid), jnp.float32),
    )(acc, inv, x, w, root, bias.reshape(1, hid))


def _decode_body(ef_ref, w1_ref, b1_ref, w2_ref, b2_ref, o_ref):
    h = jnp.dot(ef_ref[...], w1_ref[...], preferred_element_type=jnp.float32)
    h = jnp.maximum(h + b1_ref[...], 0.0)
    o_ref[...] = jnp.dot(h, w2_ref[...], preferred_element_type=jnp.float32) + b2_ref[...]


def _decode(ef, lin1_w, lin1_b, lin2_w, lin2_b):
    n, f = ef.shape
    hid = lin1_w.shape[1]
    bp = 2000
    grid = n // bp
    out = pl.pallas_call(
        _decode_body,
        grid=(grid,),
        in_specs=[
            pl.BlockSpec((bp, f), lambda i: (i, 0)),
            pl.BlockSpec((f, hid), lambda i: (0, 0)),
            pl.BlockSpec((1, hid), lambda i: (0, 0)),
            pl.BlockSpec((hid, 1), lambda i: (0, 0)),
            pl.BlockSpec((1, 1), lambda i: (0, 0)),
        ],
        out_specs=pl.BlockSpec((bp, 1), lambda i: (i, 0)),
        out_shape=jax.ShapeDtypeStruct((n, 1), jnp.float32),
    )(ef, lin1_w, lin1_b.reshape(1, hid), lin2_w, lin2_b.reshape(1, 1))
    return out[:, 0]


def kernel(edge_index, edge_type, edge_pairs, node_emb,
           basis1, comp1, root1, bias1,
           basis2, comp2, root2, bias2,
           lin1_w, lin1_b, lin2_w, lin2_b):
    src = edge_index[0]
    dst = edge_index[1]
    x = _rgcn_layer(node_emb, src, dst, edge_type, basis1, comp1, root1, bias1,
                    relu=True)
    x = _rgcn_layer(x, src, dst, edge_type, basis2, comp2, root2, bias2,
                    relu=False)
    src_z = jnp.take(x, edge_pairs[:, 0], axis=0)
    dst_z = jnp.take(x, edge_pairs[:, 1], axis=0)
    ef = jnp.concatenate([src_z, dst_z], axis=1)
    return _decode(ef, lin1_w, lin1_b, lin2_w, lin2_b)


# speedup vs baseline: 5.2047x; 1.0010x over previous
import functools

import jax
import jax.numpy as jnp
from jax.experimental import pallas as pl

_N_NODES = 50000
_N_REL = 8


def _layer_body(a_ref, inv_ref, x_ref, w_ref, root_ref, bias_ref, o_ref, *,
                relu, emb):
    a = a_ref[...]
    inv = inv_ref[...]
    a = jnp.concatenate(
        [a[:, r * emb:(r + 1) * emb] * inv[:, r:r + 1] for r in range(_N_REL)],
        axis=1)
    out = jnp.dot(a, w_ref[...], preferred_element_type=jnp.float32)
    out = out + jnp.dot(x_ref[...], root_ref[...], preferred_element_type=jnp.float32)
    out = out + bias_ref[...]
    if relu:
        out = jnp.maximum(out, 0.0)
    o_ref[...] = out


def _rgcn_layer(x, src, dst, etype, basis, comp, root, bias, relu):
    emb = x.shape[1]
    hid = basis.shape[2]
    w = jnp.einsum('rb,bio->rio', comp, basis).reshape(_N_REL * emb, hid)
    seg = dst * _N_REL + etype
    msgs = jnp.take(x, src, axis=0)
    acc = jax.ops.segment_sum(msgs, seg, num_segments=_N_NODES * _N_REL)
    cnt = jax.ops.segment_sum(jnp.ones((src.shape[0],), jnp.float32), seg,
                              num_segments=_N_NODES * _N_REL)
    acc = acc.reshape(_N_NODES, _N_REL * emb)
    inv = (1.0 / jnp.maximum(cnt, 1.0)).reshape(_N_NODES, _N_REL)

    bn = 1000
    grid = _N_NODES // bn
    return pl.pallas_call(
        functools.partial(_layer_body, relu=relu, emb=emb),
        grid=(grid,),
        in_specs=[
            pl.BlockSpec((bn, _N_REL * emb), lambda i: (i, 0)),
            pl.BlockSpec((bn, _N_REL), lambda i: (i, 0)),
            pl.BlockSpec((bn, emb), lambda i: (i, 0)),
            pl.BlockSpec((_N_REL * emb, hid), lambda i: (0, 0)),
            pl.BlockSpec((emb, hid), lambda i: (0, 0)),
            pl.BlockSpec((1, hid), lambda i: (0, 0)),
        ],
        out_specs=pl.BlockSpec((bn, hid), lambda i: (i, 0)),
        out_shape=jax.ShapeDtypeStruct((_N_NODES, hid), jnp.float32),
    )(acc, inv, x, w, root, bias.reshape(1, hid))


def _decode_body(ef_ref, w1_ref, b1_ref, w2_ref, b2_ref, o_ref):
    h = jnp.dot(ef_ref[...], w1_ref[...], preferred_element_type=jnp.float32)
    h = jnp.maximum(h + b1_ref[...], 0.0)
    o_ref[...] = jnp.dot(h, w2_ref[...], preferred_element_type=jnp.float32) + b2_ref[...]


def _decode(ef, lin1_w, lin1_b, lin2_w, lin2_b):
    n, f = ef.shape
    hid = lin1_w.shape[1]
    bp = 2000
    grid = n // bp
    out = pl.pallas_call(
        _decode_body,
        grid=(grid,),
        in_specs=[
            pl.BlockSpec((bp, f), lambda i: (i, 0)),
            pl.BlockSpec((f, hid), lambda i: (0, 0)),
            pl.BlockSpec((1, hid), lambda i: (0, 0)),
            pl.BlockSpec((hid, 1), lambda i: (0, 0)),
            pl.BlockSpec((1, 1), lambda i: (0, 0)),
        ],
        out_specs=pl.BlockSpec((bp, 1), lambda i: (i, 0)),
        out_shape=jax.ShapeDtypeStruct((n, 1), jnp.float32),
    )(ef, lin1_w, lin1_b.reshape(1, hid), lin2_w, lin2_b.reshape(1, 1))
    return out[:, 0]


def kernel(edge_index, edge_type, edge_pairs, node_emb,
           basis1, comp1, root1, bias1,
           basis2, comp2, root2, bias2,
           lin1_w, lin1_b, lin2_w, lin2_b):
    src = edge_index[0]
    dst = edge_index[1]
    x = _rgcn_layer(node_emb, src, dst, edge_type, basis1, comp1, root1, bias1,
                    relu=True)
    x = _rgcn_layer(x, src, dst, edge_type, basis2, comp2, root2, bias2,
                    relu=False)
    src_z = jnp.take(x, edge_pairs[:, 0], axis=0)
    dst_z = jnp.take(x, edge_pairs[:, 1], axis=0)
    ef = jnp.concatenate([src_z, dst_z], axis=1)
    return _decode(ef, lin1_w, lin1_b, lin2_w, lin2_b)
